# baseline (speedup 1.0000x reference)
import functools
import math

import jax
import jax.numpy as jnp
from jax import lax
from jax.experimental import pallas as pl
from jax.experimental.pallas import tpu as pltpu
from jax.experimental.pallas import tpu_sc as plsc

_NUM_CHANNELS = 64
_N_TOKENS = 32768
_D = 16
_DISC = 8
_NC = 2
_NS = 16
_NW = _NC * _NS
_L = 16
_TOK_PER_W = _N_TOKENS // _NW
_GROUPS = _TOK_PER_W // _L

_mesh = plsc.VectorSubcoreMesh(core_axis_name="c", subcore_axis_name="s")


@functools.partial(
    pl.kernel,
    out_type=jax.ShapeDtypeStruct((_N_TOKENS,), jnp.float32),
    mesh=_mesh,
    compiler_params=pltpu.CompilerParams(needs_layout_passes=False),
    scratch_types=[
        pltpu.VMEM((_D + 1, _TOK_PER_W), jnp.float32),
        pltpu.VMEM((_TOK_PER_W,), jnp.int32),
        pltpu.VMEM((_D * _NUM_CHANNELS * 4,), jnp.float32),
        pltpu.VMEM((_D * _NUM_CHANNELS * 4,), jnp.float32),
        pltpu.VMEM((_NUM_CHANNELS * _DISC * 4,), jnp.float32),
        pltpu.VMEM((_TOK_PER_W,), jnp.float32),
        pltpu.SemaphoreType.DMA,
    ],
)
def _sc_integrand(xt_hbm, ch_hbm, niv_hbm, av_hbm, tb_hbm, out_hbm,
                  xt_v, ch_v, niv_v, av_v, tb_v, out_v, sem):
    wid = lax.axis_index("s") * _NC + lax.axis_index("c")
    base = wid * _TOK_PER_W
    copies = [
        pltpu.async_copy(xt_hbm.at[:, pl.ds(base, _TOK_PER_W)], xt_v, sem),
        pltpu.async_copy(ch_hbm.at[pl.ds(base, _TOK_PER_W)], ch_v, sem),
        pltpu.async_copy(niv_hbm, niv_v, sem),
        pltpu.async_copy(av_hbm, av_v, sem),
        pltpu.async_copy(tb_hbm, tb_v, sem),
    ]
    for cp in copies:
        cp.wait()

    lane_mod = lax.iota(jnp.int32, _L) & 3

    @plsc.parallel_loop(0, _GROUPS, 1, unroll=1)
    def _group(g):
        tok = pl.ds(g * _L, _L)
        ch = ch_v[tok]
        ch4 = ch * 4 + lane_mod
        xi = jnp.clip(xt_v[_D, tok].astype(jnp.int32), 0, _DISC - 1)
        tbg = plsc.load_gather(tb_v, [(ch * _DISC + xi) * 4 + lane_mod])
        accs = []
        for d in range(_D):
            xv = xt_v[d, tok]
            nivd = plsc.load_gather(niv_v, [ch4 + d * _NUM_CHANNELS * 4])
            avd = plsc.load_gather(av_v, [ch4 + d * _NUM_CHANNELS * 4])
            term = xv * (xv * nivd + avd)
            if d < 4:
                accs.append(term)
            else:
                accs[d % 4] = accs[d % 4] + term
        out_v[tok] = jnp.exp(
            ((accs[0] + accs[1]) + (accs[2] + accs[3])) + tbg)

    pltpu.sync_copy(out_v, out_hbm.at[pl.ds(base, _TOK_PER_W)])


def kernel(x, channel, mu, log_sigma, disc_logits):
    invvar = jnp.exp(-2.0 * log_sigma)
    niv = -0.5 * invvar
    av = mu * invvar
    bias = (-0.5 * jnp.sum(mu * mu * invvar, axis=1)
            - jnp.sum(log_sigma, axis=1)
            - 0.5 * _D * math.log(2.0 * math.pi)
            - jax.nn.logsumexp(disc_logits, axis=1))
    tb = disc_logits + bias[:, None]
    xt = x.T
    return _sc_integrand(xt, channel,
                         jnp.repeat(niv.T, 4, axis=1).reshape(-1),
                         jnp.repeat(av.T, 4, axis=1).reshape(-1),
                         jnp.repeat(tb.reshape(-1), 4))


# speedup vs baseline: 1.0767x; 1.0767x over previous
import functools
import math

import jax
import jax.numpy as jnp
from jax import lax
from jax.experimental import pallas as pl
from jax.experimental.pallas import tpu as pltpu
from jax.experimental.pallas import tpu_sc as plsc

_NUM_CHANNELS = 64
_N_TOKENS = 32768
_D = 16
_DISC = 8
_NC = 2
_NS = 16
_NW = _NC * _NS
_L = 16
_TOK_PER_W = _N_TOKENS // _NW
_GROUPS = _TOK_PER_W // _L

_mesh = plsc.VectorSubcoreMesh(core_axis_name="c", subcore_axis_name="s")


@functools.partial(
    pl.kernel,
    out_type=jax.ShapeDtypeStruct((_N_TOKENS,), jnp.float32),
    mesh=_mesh,
    compiler_params=pltpu.CompilerParams(needs_layout_passes=False),
    scratch_types=[
        pltpu.VMEM((_D + 1, _TOK_PER_W), jnp.float32),
        pltpu.VMEM((_TOK_PER_W,), jnp.int32),
        pltpu.VMEM((_D, _NUM_CHANNELS), jnp.float32),
        pltpu.VMEM((_D, _NUM_CHANNELS), jnp.float32),
        pltpu.VMEM((_NUM_CHANNELS * _DISC,), jnp.float32),
        pltpu.VMEM((_TOK_PER_W,), jnp.float32),
        pltpu.SemaphoreType.DMA,
    ],
)
def _sc_integrand(xt_hbm, ch_hbm, niv_hbm, av_hbm, tb_hbm, out_hbm,
                  xt_v, ch_v, niv_v, av_v, tb_v, out_v, sem):
    wid = lax.axis_index("s") * _NC + lax.axis_index("c")
    base = wid * _TOK_PER_W
    copies = [
        pltpu.async_copy(xt_hbm.at[:, pl.ds(base, _TOK_PER_W)], xt_v, sem),
        pltpu.async_copy(ch_hbm.at[pl.ds(base, _TOK_PER_W)], ch_v, sem),
        pltpu.async_copy(niv_hbm, niv_v, sem),
        pltpu.async_copy(av_hbm, av_v, sem),
        pltpu.async_copy(tb_hbm, tb_v, sem),
    ]
    for cp in copies:
        cp.wait()

    @plsc.parallel_loop(0, _GROUPS, 1, unroll=1)
    def _group(g):
        tok = pl.ds(g * _L, _L)
        ch = ch_v[tok]
        xi = jnp.clip(xt_v[_D, tok].astype(jnp.int32), 0, _DISC - 1)
        tbg = plsc.load_gather(tb_v, [ch * _DISC + xi])
        accs = []
        for d in range(_D):
            xv = xt_v[d, tok]
            nivd = plsc.load_gather(niv_v.at[d], [ch])
            avd = plsc.load_gather(av_v.at[d], [ch])
            term = xv * (xv * nivd + avd)
            if d < 4:
                accs.append(term)
            else:
                accs[d % 4] = accs[d % 4] + term
        out_v[tok] = jnp.exp(
            ((accs[0] + accs[1]) + (accs[2] + accs[3])) + tbg)

    pltpu.sync_copy(out_v, out_hbm.at[pl.ds(base, _TOK_PER_W)])


def kernel(x, channel, mu, log_sigma, disc_logits):
    invvar = jnp.exp(-2.0 * log_sigma)
    niv = -0.5 * invvar
    av = mu * invvar
    bias = (-0.5 * jnp.sum(mu * mu * invvar, axis=1)
            - jnp.sum(log_sigma, axis=1)
            - 0.5 * _D * math.log(2.0 * math.pi)
            - jax.nn.logsumexp(disc_logits, axis=1))
    tb = disc_logits + bias[:, None]
    xt = x.T
    return _sc_integrand(xt, channel, niv.T, av.T, tb.reshape(-1))
